# baseline (speedup 1.0000x reference)
import functools

import jax
import jax.numpy as jnp
from jax import lax
from jax.experimental import pallas as pl
from jax.experimental.pallas import tpu as pltpu
from jax.experimental.pallas import tpu_sc as plsc

MAX_LEN = 200
EMBED_DIM = 64
BATCH = 16384

_PAIR_ROWS = MAX_LEN * MAX_LEN // 2
_PAIR_W = 2 * EMBED_DIM

_info = plsc.get_sparse_core_info()
_NC, _NS, _L = _info.num_cores, _info.num_subcores, _info.num_lanes
_NW = _NC * _NS
_B_PER_W = BATCH // _NW
_CHUNK = 128
_N_CHUNKS = _B_PER_W // _CHUNK


def _gather_body(pairs_hbm, match_hbm, prefix_hbm, out_hbm,
                 match_v, prefix_v, pidx_v, par_v, pairs_v, rows_v, sem):
    wid = lax.axis_index("s") * _NC + lax.axis_index("c")
    base = wid * _B_PER_W

    pltpu.sync_copy(match_hbm.at[pl.ds(base, _B_PER_W)], match_v)
    pltpu.sync_copy(prefix_hbm.at[pl.ds(base, _B_PER_W)], prefix_v)

    for i in range(_B_PER_W // _L):
        sl = pl.ds(i * _L, _L)
        flat = match_v[sl] * MAX_LEN + prefix_v[sl]
        pidx_v[sl] = lax.shift_right_logical(flat, 1)
        par_v[sl] = lax.bitwise_and(flat, 1)

    copies = []
    for j in range(_N_CHUNKS):
        sl = pl.ds(j * _CHUNK, _CHUNK)
        copies.append(pltpu.async_copy(pairs_hbm.at[pidx_v.at[sl]],
                                       pairs_v.at[sl], sem))
    for c in copies:
        c.wait()

    iota = lax.iota(jnp.int32, _L)

    def select_block(qb, _):
        row = qb * _L + iota
        par_off = par_v[pl.ds(qb * _L, _L)] * EMBED_DIM
        for c in range(EMBED_DIM):
            v = plsc.load_gather(pairs_v, [row, par_off + c])
            plsc.store_scatter(rows_v, [row, jnp.full((_L,), c, jnp.int32)], v)
        return _

    lax.fori_loop(0, _B_PER_W // _L, select_block, None)

    pltpu.sync_copy(rows_v, out_hbm.at[pl.ds(base, _B_PER_W)])


@jax.jit
def _gather(pairs_table, match_idx, prefix_idx):
    mesh = plsc.VectorSubcoreMesh(core_axis_name="c", subcore_axis_name="s")
    return pl.kernel(
        _gather_body,
        mesh=mesh,
        out_type=jax.ShapeDtypeStruct((BATCH, EMBED_DIM), jnp.float32),
        scratch_types=[
            pltpu.VMEM((_B_PER_W,), jnp.int32),
            pltpu.VMEM((_B_PER_W,), jnp.int32),
            pltpu.VMEM((_B_PER_W,), jnp.int32),
            pltpu.VMEM((_B_PER_W,), jnp.int32),
            pltpu.VMEM((_B_PER_W, _PAIR_W), jnp.float32),
            pltpu.VMEM((_B_PER_W, EMBED_DIM), jnp.float32),
            pltpu.SemaphoreType.DMA,
        ],
        compiler_params=pltpu.CompilerParams(use_tc_tiling_on_sc=False,
                                             needs_layout_passes=False),
    )(pairs_table, match_idx, prefix_idx)


def kernel(table, match_len_idx, prefix_len_idx):
    pairs_table = table.reshape(_PAIR_ROWS, _PAIR_W)
    return _gather(pairs_table,
                   match_len_idx.astype(jnp.int32),
                   prefix_len_idx.astype(jnp.int32))


# speedup vs baseline: 1.5750x; 1.5750x over previous
import functools

import jax
import jax.numpy as jnp
from jax import lax
from jax.experimental import pallas as pl
from jax.experimental.pallas import tpu as pltpu
from jax.experimental.pallas import tpu_sc as plsc

MAX_LEN = 200
EMBED_DIM = 64
BATCH = 16384

_info = plsc.get_sparse_core_info()
_NC, _NS, _L = _info.num_cores, _info.num_subcores, _info.num_lanes
_NW = _NC * _NS
_B_PER_W = BATCH // _NW
_CHUNK = 128
_N_CHUNKS = _B_PER_W // _CHUNK


def _gather_body(table_hbm, match_hbm, prefix_hbm, out_hbm,
                 match_v, prefix_v, idx_v, rows_v, sem, wsem):
    wid = lax.axis_index("s") * _NC + lax.axis_index("c")
    base = wid * _B_PER_W
    flat_table = table_hbm

    pltpu.sync_copy(match_hbm.at[pl.ds(base, _B_PER_W)], match_v)
    pltpu.sync_copy(prefix_hbm.at[pl.ds(base, _B_PER_W)], prefix_v)

    for i in range(_B_PER_W // _L):
        sl = pl.ds(i * _L, _L)
        idx_v[sl] = match_v[sl] * MAX_LEN + prefix_v[sl]

    copies = []
    for j in range(_N_CHUNKS):
        sl = pl.ds(j * _CHUNK, _CHUNK)
        copies.append(pltpu.async_copy(flat_table.at[idx_v.at[sl]],
                                       rows_v.at[sl], sem))
    writes = []
    for j in range(_N_CHUNKS):
        sl = pl.ds(j * _CHUNK, _CHUNK)
        copies[j].wait()
        writes.append(pltpu.async_copy(rows_v.at[sl],
                                       out_hbm.at[pl.ds(base + j * _CHUNK,
                                                        _CHUNK)], wsem))
    for w in writes:
        w.wait()


@jax.jit
def _gather(table, match_idx, prefix_idx):
    mesh = plsc.VectorSubcoreMesh(core_axis_name="c", subcore_axis_name="s")
    return pl.kernel(
        _gather_body,
        mesh=mesh,
        out_type=jax.ShapeDtypeStruct((BATCH, EMBED_DIM), jnp.float32),
        scratch_types=[
            pltpu.VMEM((_B_PER_W,), jnp.int32),
            pltpu.VMEM((_B_PER_W,), jnp.int32),
            pltpu.VMEM((_B_PER_W,), jnp.int32),
            pltpu.VMEM((_B_PER_W, EMBED_DIM), jnp.float32),
            pltpu.SemaphoreType.DMA,
            pltpu.SemaphoreType.DMA,
        ],
        compiler_params=pltpu.CompilerParams(use_tc_tiling_on_sc=False),
    )(table, match_idx, prefix_idx)


def kernel(table, match_len_idx, prefix_len_idx):
    flat_table = table.reshape(MAX_LEN * MAX_LEN, EMBED_DIM)
    return _gather(flat_table,
                   match_len_idx.astype(jnp.int32),
                   prefix_len_idx.astype(jnp.int32))


# speedup vs baseline: 1.5751x; 1.0001x over previous
import functools

import jax
import jax.numpy as jnp
from jax import lax
from jax.experimental import pallas as pl
from jax.experimental.pallas import tpu as pltpu
from jax.experimental.pallas import tpu_sc as plsc

MAX_LEN = 200
EMBED_DIM = 64
BATCH = 16384

_info = plsc.get_sparse_core_info()
_NC, _NS, _L = _info.num_cores, _info.num_subcores, _info.num_lanes
_NW = _NC * _NS
_B_PER_W = BATCH // _NW
_CHUNK = 128
_N_CHUNKS = _B_PER_W // _CHUNK


def _gather_body(table_hbm, match_hbm, prefix_hbm, out_hbm,
                 match_v, prefix_v, idx_v, rows_v, sem):
    wid = lax.axis_index("s") * _NC + lax.axis_index("c")
    base = wid * _B_PER_W
    flat_table = table_hbm

    pltpu.sync_copy(match_hbm.at[pl.ds(base, _B_PER_W)], match_v)
    pltpu.sync_copy(prefix_hbm.at[pl.ds(base, _B_PER_W)], prefix_v)

    for i in range(_B_PER_W // _L):
        sl = pl.ds(i * _L, _L)
        idx_v[sl] = match_v[sl] * MAX_LEN + prefix_v[sl]

    copies = []
    for j in range(_N_CHUNKS):
        sl = pl.ds(j * _CHUNK, _CHUNK)
        copies.append(pltpu.async_copy(flat_table.at[idx_v.at[sl]],
                                       rows_v.at[sl], sem))
    for c in copies:
        c.wait()

    pltpu.sync_copy(rows_v, out_hbm.at[pl.ds(base, _B_PER_W)])


@jax.jit
def _gather(table, match_idx, prefix_idx):
    mesh = plsc.VectorSubcoreMesh(core_axis_name="c", subcore_axis_name="s")
    return pl.kernel(
        _gather_body,
        mesh=mesh,
        out_type=jax.ShapeDtypeStruct((BATCH, EMBED_DIM), jnp.float32),
        scratch_types=[
            pltpu.VMEM((_B_PER_W,), jnp.int32),
            pltpu.VMEM((_B_PER_W,), jnp.int32),
            pltpu.VMEM((_B_PER_W,), jnp.int32),
            pltpu.VMEM((_B_PER_W, EMBED_DIM), jnp.float32),
            pltpu.SemaphoreType.DMA,
        ],
        compiler_params=pltpu.CompilerParams(use_tc_tiling_on_sc=False),
    )(table, match_idx, prefix_idx)


def kernel(table, match_len_idx, prefix_len_idx):
    flat_table = table.reshape(MAX_LEN * MAX_LEN, EMBED_DIM)
    return _gather(flat_table,
                   match_len_idx.astype(jnp.int32),
                   prefix_len_idx.astype(jnp.int32))


# speedup vs baseline: 1.5786x; 1.0022x over previous
import jax
import jax.numpy as jnp
from jax import lax
from jax.experimental import pallas as pl
from jax.experimental.pallas import tpu as pltpu
from jax.experimental.pallas import tpu_sc as plsc

MAX_LEN = 200
EMBED_DIM = 64
BATCH = 16384

_info = plsc.get_sparse_core_info()
_NC, _NS, _L = _info.num_cores, _info.num_subcores, _info.num_lanes
_NW = _NC * _NS
_B_PER_W = BATCH // _NW
_CHUNK = 128
_N_CHUNKS = _B_PER_W // _CHUNK


def _gather_body(table_hbm, match_hbm, prefix_hbm, out_hbm,
                 match_v, prefix_v, idx_v, rows_v, sem):
    wid = lax.axis_index("s") * _NC + lax.axis_index("c")
    base = wid * _B_PER_W

    pltpu.sync_copy(match_hbm.at[pl.ds(base, _B_PER_W)], match_v)
    pltpu.sync_copy(prefix_hbm.at[pl.ds(base, _B_PER_W)], prefix_v)

    for i in range(_B_PER_W // _L):
        sl = pl.ds(i * _L, _L)
        idx_v[sl] = match_v[sl] * MAX_LEN + prefix_v[sl]

    copies = []
    for j in range(_N_CHUNKS):
        sl = pl.ds(j * _CHUNK, _CHUNK)
        copies.append(pltpu.async_copy(table_hbm.at[idx_v.at[sl]],
                                       rows_v.at[sl], sem))
    for c in copies:
        c.wait()

    pltpu.sync_copy(rows_v, out_hbm.at[pl.ds(base, _B_PER_W)])


@jax.jit
def _gather(table, match_idx, prefix_idx):
    mesh = plsc.VectorSubcoreMesh(core_axis_name="c", subcore_axis_name="s")
    return pl.kernel(
        _gather_body,
        mesh=mesh,
        out_type=jax.ShapeDtypeStruct((BATCH, EMBED_DIM), jnp.float32),
        scratch_types=[
            pltpu.VMEM((_B_PER_W,), jnp.int32),
            pltpu.VMEM((_B_PER_W,), jnp.int32),
            pltpu.VMEM((_B_PER_W,), jnp.int32),
            pltpu.VMEM((_B_PER_W, EMBED_DIM), jnp.float32),
            pltpu.SemaphoreType.DMA,
        ],
        compiler_params=pltpu.CompilerParams(use_tc_tiling_on_sc=False),
    )(table, match_idx, prefix_idx)


def kernel(table, match_len_idx, prefix_len_idx):
    flat_table = table.reshape(MAX_LEN * MAX_LEN, EMBED_DIM)
    return _gather(flat_table,
                   match_len_idx.astype(jnp.int32),
                   prefix_len_idx.astype(jnp.int32))
